# Initial kernel scaffold; baseline (speedup 1.0000x reference)
import jax
import jax.numpy as jnp
from jax.experimental import pallas as pl


def kernel(x, edge_index, W1, b1, W2, b2, W_out, b_out):
    raise NotImplementedError("write your pallas kernel here")


# speedup vs baseline: 11.2077x; 11.2077x over previous
import functools

import jax
import jax.numpy as jnp
from jax import lax
from jax.experimental import pallas as pl
from jax.experimental.pallas import tpu as pltpu
from jax.experimental.pallas import tpu_sc as plsc

_N = 10000
_NP = 10240
_D = 128
_DH = 64
_C = 40
_E = 320000
_EP = 327680
_CHUNK = 128
_SLABS = 16
_CPS = _EP // _SLABS // _CHUNK
_RPT = _NP // 16
_BLK = 256
_GRID = _NP // _BLK


def _zero_vmem(ref, nrows, ncols):
    zero = jnp.zeros((16,), jnp.float32)

    def body(i, _):
        for j in range(ncols // 16):
            ref[i, pl.ds(j * 16, 16)] = zero
        return 0

    lax.fori_loop(0, nrows, body, 0)


def _sc_deg_body(dst_hbm, out_hbm, idx_d, ones_v, bounce, acc):
    c = lax.axis_index("c")
    s = lax.axis_index("s")

    pltpu.sync_copy(dst_hbm.at[s].at[pl.ds(c * (_CPS // 2), _CPS // 2)], idx_d)

    one = jnp.ones((16,), jnp.float32)

    def fill(i, _):
        ones_v[i, :] = one
        return 0

    lax.fori_loop(0, _CHUNK, fill, 0)
    _zero_vmem(bounce, 128, 16)
    for r in range(_RPT // 128):
        pltpu.sync_copy(bounce, acc.at[pl.ds(s * _RPT + r * 128, 128)])
    plsc.subcore_barrier()

    def body(j, _):
        pltpu.sync_copy(ones_v, acc.at[idx_d.at[j]], add=True)
        return 0

    lax.fori_loop(0, _CPS // 2, body, 0)
    plsc.subcore_barrier()

    for r in range(_RPT // 128):
        rows = pl.ds(s * _RPT + r * 128, 128)
        pltpu.sync_copy(acc.at[rows], bounce)
        pltpu.sync_copy(bounce, out_hbm.at[c].at[rows])


def _sc_deg(dst3):
    return pl.kernel(
        _sc_deg_body,
        out_type=jax.ShapeDtypeStruct((2, _NP, 16), jnp.float32),
        mesh=plsc.VectorSubcoreMesh(core_axis_name="c", subcore_axis_name="s"),
        scratch_types=[
            pltpu.VMEM((_CPS // 2, _CHUNK), jnp.int32),
            pltpu.VMEM((_CHUNK, 16), jnp.float32),
            pltpu.VMEM((128, 16), jnp.float32),
            pltpu.VMEM_SHARED((_NP, 16), jnp.float32),
        ],
    )(dst3)


def _sc_agg_body(src_hbm, dst_hbm, ga_hbm, gb_hbm, out_hbm,
                 idx_s, idx_d, rows_a, rows_b, acc, sem_a, sem_b):
    c = lax.axis_index("c")
    s = lax.axis_index("s")

    pltpu.sync_copy(src_hbm.at[s], idx_s.at[pl.ds(0, _CPS)])
    pltpu.sync_copy(dst_hbm.at[s], idx_d)

    zero_i = jnp.zeros((16,), jnp.int32)

    def ztail(i, _):
        for j in range(_CHUNK // 16):
            idx_s[_CPS + i, pl.ds(j * 16, 16)] = zero_i
        return 0

    lax.fori_loop(0, 2, ztail, 0)

    _zero_vmem(rows_a, _CHUNK, _DH)
    for r in range(_RPT // _CHUNK):
        pltpu.sync_copy(rows_a, acc.at[pl.ds(s * _RPT + r * _CHUNK, _CHUNK)])
    plsc.subcore_barrier()

    def run(g_hbm):
        pltpu.async_copy(g_hbm.at[idx_s.at[0]], rows_a, sem_a)

        def body(jj, _):
            j = jj * 2
            pltpu.async_copy(g_hbm.at[idx_s.at[j + 1]], rows_b, sem_b)
            pltpu.make_async_copy(g_hbm.at[idx_s.at[j]], rows_a, sem_a).wait()
            pltpu.sync_copy(rows_a, acc.at[idx_d.at[j]], add=True)
            pltpu.async_copy(g_hbm.at[idx_s.at[j + 2]], rows_a, sem_a)
            pltpu.make_async_copy(g_hbm.at[idx_s.at[j + 1]], rows_b, sem_b).wait()
            pltpu.sync_copy(rows_b, acc.at[idx_d.at[j + 1]], add=True)
            return 0

        lax.fori_loop(0, _CPS // 2, body, 0)
        pltpu.make_async_copy(g_hbm.at[idx_s.at[_CPS]], rows_a, sem_a).wait()

    @pl.when(c == 0)
    def _():
        run(ga_hbm)

    @pl.when(c == 1)
    def _():
        run(gb_hbm)

    plsc.subcore_barrier()

    for r in range(_RPT // _CHUNK):
        rows = pl.ds(s * _RPT + r * _CHUNK, _CHUNK)
        pltpu.sync_copy(acc.at[rows], rows_a)
        pltpu.sync_copy(rows_a, out_hbm.at[c].at[rows])


def _sc_agg(src3, dst3, ga, gb):
    return pl.kernel(
        _sc_agg_body,
        out_type=jax.ShapeDtypeStruct((2, _NP, _DH), jnp.float32),
        mesh=plsc.VectorSubcoreMesh(core_axis_name="c", subcore_axis_name="s"),
        compiler_params=pltpu.CompilerParams(use_tc_tiling_on_sc=False),
        scratch_types=[
            pltpu.VMEM((_CPS + 2, _CHUNK), jnp.int32),
            pltpu.VMEM((_CPS, _CHUNK), jnp.int32),
            pltpu.VMEM((_CHUNK, _DH), jnp.float32),
            pltpu.VMEM((_CHUNK, _DH), jnp.float32),
            pltpu.VMEM_SHARED((_NP, _DH), jnp.float32),
            pltpu.SemaphoreType.DMA,
            pltpu.SemaphoreType.DMA,
        ],
    )(src3, dst3, ga, gb)


def _tc1_body(degp_ref, x_ref, w1_ref, g1_ref, dinv_ref):
    i = pl.program_id(0)
    deg = degp_ref[0, :, 0:1] + degp_ref[1, :, 0:1] + 1.0
    dinv = lax.rsqrt(deg)
    rows = i * _BLK + lax.broadcasted_iota(jnp.int32, (_BLK, 1), 0)
    dinv = jnp.where(rows < _N, dinv, 0.0)
    u = jnp.dot(x_ref[...], w1_ref[...], preferred_element_type=jnp.float32)
    g1_ref[...] = u * dinv
    dinv_ref[...] = jnp.broadcast_to(dinv, (_BLK, _D))


def _tc1(degp, xp, W1):
    return pl.pallas_call(
        _tc1_body,
        grid=(_GRID,),
        in_specs=[
            pl.BlockSpec((2, _BLK, 16), lambda i: (0, i, 0)),
            pl.BlockSpec((_BLK, _D), lambda i: (i, 0)),
            pl.BlockSpec((_D, _D), lambda i: (0, 0)),
        ],
        out_specs=[
            pl.BlockSpec((_BLK, _D), lambda i: (i, 0)),
            pl.BlockSpec((_BLK, _D), lambda i: (i, 0)),
        ],
        out_shape=[
            jax.ShapeDtypeStruct((_NP, _D), jnp.float32),
            jax.ShapeDtypeStruct((_NP, _D), jnp.float32),
        ],
    )(degp, xp, W1)


def _tc2_body(pa_ref, pb_ref, g1_ref, dinv_ref, w2_ref, b1_ref, g2_ref):
    p = jnp.concatenate([pa_ref[0], pb_ref[0]], axis=1)
    h = dinv_ref[...] * (p + g1_ref[...]) + b1_ref[...]
    u = jnp.dot(h, w2_ref[...], preferred_element_type=jnp.float32)
    g2_ref[...] = u * dinv_ref[...]


def _tc2(p, g1, dinv_b, W2, b1):
    pa = p[0:1]
    pb = p[1:2]
    return pl.pallas_call(
        _tc2_body,
        grid=(_GRID,),
        in_specs=[
            pl.BlockSpec((1, _BLK, _DH), lambda i: (0, i, 0)),
            pl.BlockSpec((1, _BLK, _DH), lambda i: (0, i, 0)),
            pl.BlockSpec((_BLK, _D), lambda i: (i, 0)),
            pl.BlockSpec((_BLK, _D), lambda i: (i, 0)),
            pl.BlockSpec((_D, _D), lambda i: (0, 0)),
            pl.BlockSpec((1, _D), lambda i: (0, 0)),
        ],
        out_specs=pl.BlockSpec((_BLK, _D), lambda i: (i, 0)),
        out_shape=jax.ShapeDtypeStruct((_NP, _D), jnp.float32),
    )(pa, pb, g1, dinv_b, W2, b1)


def _tc3_body(pa_ref, pb_ref, g2_ref, dinv_ref, b2_ref, wo_ref, bo_ref, o_ref):
    p = jnp.concatenate([pa_ref[0], pb_ref[0]], axis=1)
    h = dinv_ref[...] * (p + g2_ref[...]) + b2_ref[...]
    o = jnp.dot(h, wo_ref[...], preferred_element_type=jnp.float32) + bo_ref[...]
    o_ref[...] = jnp.clip(o, -4.0, 4.0)


def _tc3(p, g2, dinv_b, b2, Wo, bo):
    pa = p[0:1]
    pb = p[1:2]
    return pl.pallas_call(
        _tc3_body,
        grid=(_GRID,),
        in_specs=[
            pl.BlockSpec((1, _BLK, _DH), lambda i: (0, i, 0)),
            pl.BlockSpec((1, _BLK, _DH), lambda i: (0, i, 0)),
            pl.BlockSpec((_BLK, _D), lambda i: (i, 0)),
            pl.BlockSpec((_BLK, _D), lambda i: (i, 0)),
            pl.BlockSpec((1, _D), lambda i: (0, 0)),
            pl.BlockSpec((_D, _D), lambda i: (0, 0)),
            pl.BlockSpec((1, _D), lambda i: (0, 0)),
        ],
        out_specs=pl.BlockSpec((_BLK, _D), lambda i: (i, 0)),
        out_shape=jax.ShapeDtypeStruct((_NP, _D), jnp.float32),
    )(pa, pb, g2, dinv_b, b2, Wo, bo)


@jax.jit
def kernel(x, edge_index, W1, b1, W2, b2, W_out, b_out):
    src = edge_index[0]
    dst = edge_index[1]
    pad = _EP - _E
    src3 = jnp.concatenate([src, jnp.zeros((pad,), jnp.int32)]).reshape(
        _SLABS, _CPS, _CHUNK)
    dst3 = jnp.concatenate([dst, jnp.full((pad,), _N, jnp.int32)]).reshape(
        _SLABS, _CPS, _CHUNK)
    xp = jnp.pad(x, ((0, _NP - _N), (0, 0)))
    b1r = b1.reshape(1, _D)
    b2r = b2.reshape(1, _D)
    Wo = jnp.pad(W_out, ((0, 0), (0, _D - _C)))
    bo = jnp.pad(b_out, (0, _D - _C)).reshape(1, _D)

    degp = _sc_deg(dst3)
    g1, dinv_b = _tc1(degp, xp, W1)
    s1 = _sc_agg(src3, dst3, g1[:, :_DH], g1[:, _DH:])
    g2 = _tc2(s1, g1, dinv_b, W2, b1r)
    s2 = _sc_agg(src3, dst3, g2[:, :_DH], g2[:, _DH:])
    out = _tc3(s2, g2, dinv_b, b2r, Wo, bo)
    return out[:_N, :_C]
